# baseline (speedup 1.0000x reference)
import functools

import jax
import jax.numpy as jnp
from jax import lax
from jax.experimental import pallas as pl
from jax.experimental.pallas import tpu as pltpu
from jax.experimental.pallas import tpu_sc as plsc

N = 10000
E = 320000
D = 128

NC = 2
NS = 16
NW = NC * NS
EPW = E // NW
CB = 80
NCHUNK = EPW // CB
NP = 10240
RPT = NP // NS


def _sc_aggregate(x, src3, dst3, zeros1, ones):
  mesh = plsc.VectorSubcoreMesh(core_axis_name="c", subcore_axis_name="s")

  @functools.partial(
      pl.kernel,
      out_type=(jax.ShapeDtypeStruct((NC, NP, D), jnp.float32),
                jax.ShapeDtypeStruct((NC, NP, 8), jnp.float32)),
      mesh=mesh,
      compiler_params=pltpu.CompilerParams(use_tc_tiling_on_sc=False,
                                           needs_layout_passes=False),
      scratch_types=[
          pltpu.VMEM((NCHUNK, CB), jnp.int32),
          pltpu.VMEM((NCHUNK, CB), jnp.int32),
          pltpu.VMEM((CB, D), jnp.float32),
          pltpu.VMEM((CB, D), jnp.float32),
          pltpu.VMEM((CB, 8), jnp.float32),
          pltpu.VMEM_SHARED((NP, D), jnp.float32),
          pltpu.VMEM_SHARED((NP, 8), jnp.float32),
          pltpu.SemaphoreType.DMA,
          pltpu.SemaphoreType.DMA,
          pltpu.SemaphoreType.DMA,
      ],
  )
  def agg_kernel(x_hbm, src_hbm, dst_hbm, z1_hbm, o_hbm, acc_hbm, deg_hbm,
                 src_v, dst_v, gbufa, gbufb, ones_v, acc_sh, deg_sh,
                 sema, semb, semd):
    cid = lax.axis_index("c")
    sid = lax.axis_index("s")
    wid = cid * NS + sid

    pltpu.sync_copy(src_hbm.at[wid], src_v)
    pltpu.sync_copy(dst_hbm.at[wid], dst_v)

    z16 = jnp.zeros((16,), jnp.float32)

    @pl.loop(0, CB)
    def _(r):
      for c in range(D // 16):
        gbufa[r, pl.ds(c * 16, 16)] = z16

    for k in range(RPT // CB):
      pltpu.sync_copy(gbufa, acc_sh.at[pl.ds(sid * RPT + k * CB, CB)])

    pltpu.sync_copy(z1_hbm.at[pl.ds(sid * RPT, RPT)],
                    deg_sh.at[pl.ds(sid * RPT, RPT)])
    pltpu.sync_copy(o_hbm, ones_v)
    plsc.subcore_barrier()

    def scatter(gbuf, j):
      pltpu.sync_copy(gbuf, acc_sh.at[dst_v.at[j]], add=True)
      pltpu.async_copy(ones_v, deg_sh.at[dst_v.at[j]], semd, add=True)

    pltpu.async_copy(x_hbm.at[src_v.at[0]], gbufa, sema)

    @pl.loop(0, NCHUNK - 1, step=2)
    def _(j):
      hb = pltpu.async_copy(x_hbm.at[src_v.at[j + 1]], gbufb, semb)
      pltpu.make_async_copy(x_hbm.at[pl.ds(0, CB)], gbufa, sema).wait()
      scatter(gbufa, j)

      @pl.when(j + 2 < NCHUNK)
      def _():
        pltpu.async_copy(x_hbm.at[src_v.at[j + 2]], gbufa, sema)

      hb.wait()
      scatter(gbufb, j + 1)

    pltpu.make_async_copy(x_hbm.at[pl.ds(0, CB)], gbufa, sema).wait()
    scatter(gbufa, NCHUNK - 1)

    @pl.loop(0, NCHUNK)
    def _(j):
      pltpu.make_async_copy(z1_hbm.at[pl.ds(0, CB)], ones_v, semd).wait()

    plsc.subcore_barrier()

    pltpu.sync_copy(acc_sh.at[pl.ds(sid * RPT, RPT)],
                    acc_hbm.at[cid, pl.ds(sid * RPT, RPT)])
    pltpu.sync_copy(deg_sh.at[pl.ds(sid * RPT, RPT)],
                    deg_hbm.at[cid, pl.ds(sid * RPT, RPT)])

  return agg_kernel(x, src3, dst3, zeros1, ones)


def _tc_root_body(x_ref, wr_ref, bl_ref, out_ref):
  out_ref[...] = lax.dot_general(
      x_ref[...], wr_ref[...], (((1,), (1,)), ((), ())),
      preferred_element_type=jnp.float32) + bl_ref[...]


def _tc_root(x, W_r, b_l2):
  blk = 2000
  return pl.pallas_call(
      _tc_root_body,
      grid=(N // blk,),
      in_specs=[
          pl.BlockSpec((blk, D), lambda i: (i, 0)),
          pl.BlockSpec((D, D), lambda i: (0, 0)),
          pl.BlockSpec((1, D), lambda i: (0, 0)),
      ],
      out_specs=pl.BlockSpec((blk, D), lambda i: (i, 0)),
      out_shape=jax.ShapeDtypeStruct((N, D), jnp.float32),
  )(x, W_r, b_l2)


def _tc_finish_body(agg_ref, deg_ref, xr_ref, wl_ref, out_ref):
  a = agg_ref[0] + agg_ref[1]
  deg = jnp.sum(deg_ref[0] + deg_ref[1], axis=-1, keepdims=True)
  mean = a / jnp.maximum(deg, 1.0)
  out = lax.dot_general(mean, wl_ref[...], (((1,), (1,)), ((), ())),
                        preferred_element_type=jnp.float32) + xr_ref[...]
  norm = jnp.sqrt(jnp.sum(out * out, axis=-1, keepdims=True))
  out_ref[...] = out / jnp.maximum(norm, 1e-12)


def _tc_finish(agg2, deg2, xr, W_l):
  blk = 2000
  return pl.pallas_call(
      _tc_finish_body,
      grid=(N // blk,),
      in_specs=[
          pl.BlockSpec((NC, blk, D), lambda i: (0, i, 0)),
          pl.BlockSpec((NC, blk, 8), lambda i: (0, i, 0)),
          pl.BlockSpec((blk, D), lambda i: (i, 0)),
          pl.BlockSpec((D, D), lambda i: (0, 0)),
      ],
      out_specs=pl.BlockSpec((blk, D), lambda i: (i, 0)),
      out_shape=jax.ShapeDtypeStruct((N, D), jnp.float32),
  )(agg2, deg2, xr, W_l)


@jax.jit
def kernel(x, edge_index, W_l, b_l, W_r):
  ei4 = edge_index.reshape(2, NW, NCHUNK, CB)
  zeros1 = jnp.zeros((NP, 8), jnp.float32)
  ones = jnp.zeros((CB, 8), jnp.float32).at[:, 0].set(1.0)
  xr = _tc_root(x, W_r, b_l.reshape(1, D))
  agg2, deg2 = _sc_aggregate(x, ei4[0], ei4[1], zeros1, ones)
  return _tc_finish(agg2, deg2, xr, W_l)


# speedup vs baseline: 1.0001x; 1.0001x over previous
import functools

import jax
import jax.numpy as jnp
from jax import lax
from jax.experimental import pallas as pl
from jax.experimental.pallas import tpu as pltpu
from jax.experimental.pallas import tpu_sc as plsc

N = 10000
E = 320000
D = 128

NC = 2
NS = 16
NW = NC * NS
EPW = E // NW
CB = 80
NCHUNK = EPW // CB
NP = 10240
RPT = NP // NS


def _sc_aggregate(x, src3, dst3, zeros1, ones):
  mesh = plsc.VectorSubcoreMesh(core_axis_name="c", subcore_axis_name="s")

  @functools.partial(
      pl.kernel,
      out_type=(jax.ShapeDtypeStruct((NC, NP, D), jnp.float32),
                jax.ShapeDtypeStruct((NC, NP, 8), jnp.float32)),
      mesh=mesh,
      compiler_params=pltpu.CompilerParams(use_tc_tiling_on_sc=False,
                                           needs_layout_passes=False),
      scratch_types=[
          pltpu.VMEM((NCHUNK, CB), jnp.int32),
          pltpu.VMEM((NCHUNK, CB), jnp.int32),
          pltpu.VMEM((CB, D), jnp.float32),
          pltpu.VMEM((CB, D), jnp.float32),
          pltpu.VMEM((CB, 8), jnp.float32),
          pltpu.VMEM_SHARED((NP, D), jnp.float32),
          pltpu.VMEM_SHARED((NP, 8), jnp.float32),
          pltpu.SemaphoreType.DMA,
          pltpu.SemaphoreType.DMA,
          pltpu.SemaphoreType.DMA,
      ],
  )
  def agg_kernel(x_hbm, src_hbm, dst_hbm, z1_hbm, o_hbm, acc_hbm, deg_hbm,
                 src_v, dst_v, gbufa, gbufb, ones_v, acc_sh, deg_sh,
                 sema, semb, semd):
    cid = lax.axis_index("c")
    sid = lax.axis_index("s")
    wid = cid * NS + sid

    pltpu.sync_copy(src_hbm.at[wid], src_v)
    pltpu.sync_copy(dst_hbm.at[wid], dst_v)

    z16 = jnp.zeros((16,), jnp.float32)

    @pl.loop(0, CB)
    def _(r):
      for c in range(D // 16):
        gbufa[r, pl.ds(c * 16, 16)] = z16

    for k in range(RPT // CB):
      pltpu.sync_copy(gbufa, acc_sh.at[pl.ds(sid * RPT + k * CB, CB)])

    pltpu.sync_copy(z1_hbm.at[pl.ds(sid * RPT, RPT)],
                    deg_sh.at[pl.ds(sid * RPT, RPT)])
    pltpu.sync_copy(o_hbm, ones_v)
    plsc.subcore_barrier()

    def scatter(gbuf, j):
      pltpu.sync_copy(gbuf, acc_sh.at[dst_v.at[j]], add=True)
      pltpu.async_copy(ones_v, deg_sh.at[dst_v.at[j]], semd, add=True)

    pltpu.async_copy(x_hbm.at[src_v.at[0]], gbufa, sema)

    @pl.loop(0, NCHUNK - 1, step=2)
    def _(j):
      hb = pltpu.async_copy(x_hbm.at[src_v.at[j + 1]], gbufb, semb)
      pltpu.make_async_copy(x_hbm.at[pl.ds(0, CB)], gbufa, sema).wait()
      scatter(gbufa, j)

      @pl.when(j + 2 < NCHUNK)
      def _():
        pltpu.async_copy(x_hbm.at[src_v.at[j + 2]], gbufa, sema)

      hb.wait()
      scatter(gbufb, j + 1)

    pltpu.make_async_copy(x_hbm.at[pl.ds(0, CB)], gbufa, sema).wait()
    scatter(gbufa, NCHUNK - 1)

    @pl.loop(0, NCHUNK)
    def _(j):
      pltpu.make_async_copy(z1_hbm.at[pl.ds(0, CB)], ones_v, semd).wait()

    plsc.subcore_barrier()

    pltpu.sync_copy(acc_sh.at[pl.ds(sid * RPT, RPT)],
                    acc_hbm.at[cid, pl.ds(sid * RPT, RPT)])
    pltpu.sync_copy(deg_sh.at[pl.ds(sid * RPT, RPT)],
                    deg_hbm.at[cid, pl.ds(sid * RPT, RPT)])

  return agg_kernel(x, src3, dst3, zeros1, ones)


def _tc_root_body(x_ref, wr_ref, bl_ref, out_ref):
  out_ref[...] = lax.dot_general(
      x_ref[...], wr_ref[...], (((1,), (1,)), ((), ())),
      preferred_element_type=jnp.float32) + bl_ref[...]


def _tc_root(x, W_r, b_l2):
  blk = 2000
  return pl.pallas_call(
      _tc_root_body,
      grid=(N // blk,),
      in_specs=[
          pl.BlockSpec((blk, D), lambda i: (i, 0)),
          pl.BlockSpec((D, D), lambda i: (0, 0)),
          pl.BlockSpec((1, D), lambda i: (0, 0)),
      ],
      out_specs=pl.BlockSpec((blk, D), lambda i: (i, 0)),
      out_shape=jax.ShapeDtypeStruct((N, D), jnp.float32),
  )(x, W_r, b_l2)


def _tc_finish_body(agg_ref, deg_ref, x_ref, wl_ref, bl_ref, wr_ref, out_ref):
  a = agg_ref[0] + agg_ref[1]
  deg = jnp.sum(deg_ref[0] + deg_ref[1], axis=-1, keepdims=True)
  mean = a / jnp.maximum(deg, 1.0)
  out = (
      lax.dot_general(mean, wl_ref[...], (((1,), (1,)), ((), ())),
                      preferred_element_type=jnp.float32)
      + lax.dot_general(x_ref[...], wr_ref[...], (((1,), (1,)), ((), ())),
                        preferred_element_type=jnp.float32)
      + bl_ref[...]
  )
  norm = jnp.sqrt(jnp.sum(out * out, axis=-1, keepdims=True))
  out_ref[...] = out / jnp.maximum(norm, 1e-12)


def _tc_finish(agg2, deg2, x, W_l, b_l2, W_r):
  blk = 2000
  return pl.pallas_call(
      _tc_finish_body,
      grid=(N // blk,),
      in_specs=[
          pl.BlockSpec((NC, blk, D), lambda i: (0, i, 0)),
          pl.BlockSpec((NC, blk, 8), lambda i: (0, i, 0)),
          pl.BlockSpec((blk, D), lambda i: (i, 0)),
          pl.BlockSpec((D, D), lambda i: (0, 0)),
          pl.BlockSpec((1, D), lambda i: (0, 0)),
          pl.BlockSpec((D, D), lambda i: (0, 0)),
      ],
      out_specs=pl.BlockSpec((blk, D), lambda i: (i, 0)),
      out_shape=jax.ShapeDtypeStruct((N, D), jnp.float32),
  )(agg2, deg2, x, W_l, b_l2, W_r)


@jax.jit
def kernel(x, edge_index, W_l, b_l, W_r):
  ei4 = edge_index.reshape(2, NW, NCHUNK, CB)
  zeros1 = jnp.zeros((NP, 8), jnp.float32)
  ones = jnp.zeros((CB, 8), jnp.float32).at[:, 0].set(1.0)
  agg2, deg2 = _sc_aggregate(x, ei4[0], ei4[1], zeros1, ones)
  return _tc_finish(agg2, deg2, x, W_l, b_l.reshape(1, D), W_r)


# speedup vs baseline: 1.0086x; 1.0085x over previous
import functools

import jax
import jax.numpy as jnp
from jax import lax
from jax.experimental import pallas as pl
from jax.experimental.pallas import tpu as pltpu
from jax.experimental.pallas import tpu_sc as plsc

N = 10000
E = 320000
D = 128

NC = 2
NS = 16
NW = NC * NS
EPW = E // NW
EPW_PAD = 10240
CB = 80
NCHUNK = EPW_PAD // CB
NP = 10240
RPT = NP // NS


def _sc_aggregate(x, src3, dst3, zeros, zeros1, ones):
  mesh = plsc.VectorSubcoreMesh(core_axis_name="c", subcore_axis_name="s")

  @functools.partial(
      pl.kernel,
      out_type=(jax.ShapeDtypeStruct((NC, NP, D), jnp.float32),
                jax.ShapeDtypeStruct((NC, NP, 8), jnp.float32)),
      mesh=mesh,
      compiler_params=pltpu.CompilerParams(use_tc_tiling_on_sc=False,
                                           needs_layout_passes=False),
      scratch_types=[
          pltpu.VMEM((NCHUNK, CB), jnp.int32),
          pltpu.VMEM((NCHUNK, CB), jnp.int32),
          pltpu.VMEM((CB, D), jnp.float32),
          pltpu.VMEM((CB, D), jnp.float32),
          pltpu.VMEM((CB, 8), jnp.float32),
          pltpu.VMEM_SHARED((NP, D), jnp.float32),
          pltpu.VMEM_SHARED((NP, 8), jnp.float32),
          pltpu.SemaphoreType.DMA,
          pltpu.SemaphoreType.DMA,
          pltpu.SemaphoreType.DMA,
      ],
  )
  def agg_kernel(x_hbm, src_hbm, dst_hbm, z_hbm, z1_hbm, o_hbm, acc_hbm, deg_hbm,
                 src_v, dst_v, gbufa, gbufb, ones_v, acc_sh, deg_sh,
                 sema, semb, semd):
    cid = lax.axis_index("c")
    sid = lax.axis_index("s")
    wid = cid * NS + sid

    pltpu.sync_copy(src_hbm.at[wid], src_v)
    pltpu.sync_copy(dst_hbm.at[wid], dst_v)

    pltpu.sync_copy(z_hbm.at[pl.ds(sid * RPT, RPT)],
                    acc_sh.at[pl.ds(sid * RPT, RPT)])
    pltpu.sync_copy(z1_hbm.at[pl.ds(sid * RPT, RPT)],
                    deg_sh.at[pl.ds(sid * RPT, RPT)])
    pltpu.sync_copy(o_hbm, ones_v)
    plsc.subcore_barrier()

    def scatter(gbuf, j):
      pltpu.sync_copy(gbuf, acc_sh.at[dst_v.at[j]], add=True)
      pltpu.async_copy(ones_v, deg_sh.at[dst_v.at[j]], semd, add=True)

    pltpu.async_copy(x_hbm.at[src_v.at[0]], gbufa, sema)

    @pl.loop(0, NCHUNK, step=2)
    def _(j):
      hb = pltpu.async_copy(x_hbm.at[src_v.at[j + 1]], gbufb, semb)
      pltpu.make_async_copy(x_hbm.at[pl.ds(0, CB)], gbufa, sema).wait()
      scatter(gbufa, j)

      @pl.when(j + 2 < NCHUNK)
      def _():
        pltpu.async_copy(x_hbm.at[src_v.at[j + 2]], gbufa, sema)

      hb.wait()
      scatter(gbufb, j + 1)

    @pl.loop(0, NCHUNK)
    def _(j):
      pltpu.make_async_copy(z1_hbm.at[pl.ds(0, CB)], ones_v, semd).wait()

    plsc.subcore_barrier()

    pltpu.sync_copy(acc_sh.at[pl.ds(sid * RPT, RPT)],
                    acc_hbm.at[cid, pl.ds(sid * RPT, RPT)])
    pltpu.sync_copy(deg_sh.at[pl.ds(sid * RPT, RPT)],
                    deg_hbm.at[cid, pl.ds(sid * RPT, RPT)])

  return agg_kernel(x, src3, dst3, zeros, zeros1, ones)


def _tc_finish_body(agg_ref, deg_ref, x_ref, wl_ref, bl_ref, wr_ref, out_ref):
  a = agg_ref[0] + agg_ref[1]
  deg = jnp.sum(deg_ref[0] + deg_ref[1], axis=-1, keepdims=True)
  mean = a / jnp.maximum(deg, 1.0)
  out = (
      lax.dot_general(mean, wl_ref[...], (((1,), (1,)), ((), ())),
                      preferred_element_type=jnp.float32)
      + lax.dot_general(x_ref[...], wr_ref[...], (((1,), (1,)), ((), ())),
                        preferred_element_type=jnp.float32)
      + bl_ref[...]
  )
  norm = jnp.sqrt(jnp.sum(out * out, axis=-1, keepdims=True))
  out_ref[...] = out / jnp.maximum(norm, 1e-12)


def _tc_finish(agg2, deg2, x, W_l, b_l2, W_r):
  blk = 2000
  grid = N // blk
  return pl.pallas_call(
      _tc_finish_body,
      grid=(grid,),
      in_specs=[
          pl.BlockSpec((NC, blk, D), lambda i: (0, i, 0)),
          pl.BlockSpec((NC, blk, 8), lambda i: (0, i, 0)),
          pl.BlockSpec((blk, D), lambda i: (i, 0)),
          pl.BlockSpec((D, D), lambda i: (0, 0)),
          pl.BlockSpec((1, D), lambda i: (0, 0)),
          pl.BlockSpec((D, D), lambda i: (0, 0)),
      ],
      out_specs=pl.BlockSpec((blk, D), lambda i: (i, 0)),
      out_shape=jax.ShapeDtypeStruct((N, D), jnp.float32),
  )(agg2, deg2, x, W_l, b_l2, W_r)


@jax.jit
def kernel(x, edge_index, W_l, b_l, W_r):
  pad_src = jnp.broadcast_to(jnp.arange(EPW_PAD - EPW, dtype=jnp.int32),
                             (NW, EPW_PAD - EPW))
  src = jnp.concatenate([edge_index[0].reshape(NW, EPW), pad_src], axis=1)
  pad_dst = jnp.broadcast_to(N + jnp.arange(EPW_PAD - EPW, dtype=jnp.int32),
                             (NW, EPW_PAD - EPW))
  dst = jnp.concatenate([edge_index[1].reshape(NW, EPW), pad_dst], axis=1)
  src3 = src.reshape(NW, NCHUNK, CB)
  dst3 = dst.reshape(NW, NCHUNK, CB)
  zeros = jnp.zeros((NP, D), jnp.float32)
  zeros1 = jnp.zeros((NP, 8), jnp.float32)
  ones = jnp.zeros((CB, 8), jnp.float32).at[:, 0].set(1.0)
  agg2, deg2 = _sc_aggregate(x, src3, dst3, zeros, zeros1, ones)
  return _tc_finish(agg2, deg2, x, W_l, b_l.reshape(1, D), W_r)


# speedup vs baseline: 1.0366x; 1.0277x over previous
import functools

import jax
import jax.numpy as jnp
from jax import lax
from jax.experimental import pallas as pl
from jax.experimental.pallas import tpu as pltpu
from jax.experimental.pallas import tpu_sc as plsc

N = 10000
E = 320000
D = 128

NC = 2
NS = 16
NW = NC * NS
EPW = E // NW
EPW_PAD = 10240
CB = 80
NCHUNK = EPW_PAD // CB
NP = 10240
RPT = NP // NS


def _sc_aggregate(x, src3, dst3, zeros1, ones):
  mesh = plsc.VectorSubcoreMesh(core_axis_name="c", subcore_axis_name="s")

  @functools.partial(
      pl.kernel,
      out_type=(jax.ShapeDtypeStruct((NC, NP, D), jnp.float32),
                jax.ShapeDtypeStruct((NC, NP, 8), jnp.float32)),
      mesh=mesh,
      compiler_params=pltpu.CompilerParams(use_tc_tiling_on_sc=False,
                                           needs_layout_passes=False),
      scratch_types=[
          pltpu.VMEM((NCHUNK, CB), jnp.int32),
          pltpu.VMEM((NCHUNK, CB), jnp.int32),
          pltpu.VMEM((CB, D), jnp.float32),
          pltpu.VMEM((CB, D), jnp.float32),
          pltpu.VMEM((CB, 8), jnp.float32),
          pltpu.VMEM_SHARED((NP, D), jnp.float32),
          pltpu.VMEM_SHARED((NP, 8), jnp.float32),
          pltpu.SemaphoreType.DMA,
          pltpu.SemaphoreType.DMA,
          pltpu.SemaphoreType.DMA,
      ],
  )
  def agg_kernel(x_hbm, src_hbm, dst_hbm, z1_hbm, o_hbm, acc_hbm, deg_hbm,
                 src_v, dst_v, gbufa, gbufb, ones_v, acc_sh, deg_sh,
                 sema, semb, semd):
    cid = lax.axis_index("c")
    sid = lax.axis_index("s")
    wid = cid * NS + sid

    pltpu.sync_copy(src_hbm.at[wid], src_v)
    pltpu.sync_copy(dst_hbm.at[wid], dst_v)

    z16 = jnp.zeros((16,), jnp.float32)

    @pl.loop(0, CB)
    def _(r):
      for c in range(D // 16):
        gbufa[r, pl.ds(c * 16, 16)] = z16

    for k in range(RPT // CB):
      pltpu.sync_copy(gbufa, acc_sh.at[pl.ds(sid * RPT + k * CB, CB)])

    pltpu.sync_copy(z1_hbm.at[pl.ds(sid * RPT, RPT)],
                    deg_sh.at[pl.ds(sid * RPT, RPT)])
    pltpu.sync_copy(o_hbm, ones_v)
    plsc.subcore_barrier()

    def scatter(gbuf, j):
      pltpu.sync_copy(gbuf, acc_sh.at[dst_v.at[j]], add=True)
      pltpu.async_copy(ones_v, deg_sh.at[dst_v.at[j]], semd, add=True)

    pltpu.async_copy(x_hbm.at[src_v.at[0]], gbufa, sema)

    @pl.loop(0, NCHUNK, step=2)
    def _(j):
      hb = pltpu.async_copy(x_hbm.at[src_v.at[j + 1]], gbufb, semb)
      pltpu.make_async_copy(x_hbm.at[pl.ds(0, CB)], gbufa, sema).wait()
      scatter(gbufa, j)

      @pl.when(j + 2 < NCHUNK)
      def _():
        pltpu.async_copy(x_hbm.at[src_v.at[j + 2]], gbufa, sema)

      hb.wait()
      scatter(gbufb, j + 1)

    @pl.loop(0, NCHUNK)
    def _(j):
      pltpu.make_async_copy(z1_hbm.at[pl.ds(0, CB)], ones_v, semd).wait()

    plsc.subcore_barrier()

    pltpu.sync_copy(acc_sh.at[pl.ds(sid * RPT, RPT)],
                    acc_hbm.at[cid, pl.ds(sid * RPT, RPT)])
    pltpu.sync_copy(deg_sh.at[pl.ds(sid * RPT, RPT)],
                    deg_hbm.at[cid, pl.ds(sid * RPT, RPT)])

  return agg_kernel(x, src3, dst3, zeros1, ones)


def _tc_finish_body(agg_ref, deg_ref, x_ref, wl_ref, bl_ref, wr_ref, out_ref):
  a = agg_ref[0] + agg_ref[1]
  deg = jnp.sum(deg_ref[0] + deg_ref[1], axis=-1, keepdims=True)
  mean = a / jnp.maximum(deg, 1.0)
  out = (
      lax.dot_general(mean, wl_ref[...], (((1,), (1,)), ((), ())),
                      preferred_element_type=jnp.float32)
      + lax.dot_general(x_ref[...], wr_ref[...], (((1,), (1,)), ((), ())),
                        preferred_element_type=jnp.float32)
      + bl_ref[...]
  )
  norm = jnp.sqrt(jnp.sum(out * out, axis=-1, keepdims=True))
  out_ref[...] = out / jnp.maximum(norm, 1e-12)


def _tc_finish(agg2, deg2, x, W_l, b_l2, W_r):
  blk = 2000
  grid = N // blk
  return pl.pallas_call(
      _tc_finish_body,
      grid=(grid,),
      in_specs=[
          pl.BlockSpec((NC, blk, D), lambda i: (0, i, 0)),
          pl.BlockSpec((NC, blk, 8), lambda i: (0, i, 0)),
          pl.BlockSpec((blk, D), lambda i: (i, 0)),
          pl.BlockSpec((D, D), lambda i: (0, 0)),
          pl.BlockSpec((1, D), lambda i: (0, 0)),
          pl.BlockSpec((D, D), lambda i: (0, 0)),
      ],
      out_specs=pl.BlockSpec((blk, D), lambda i: (i, 0)),
      out_shape=jax.ShapeDtypeStruct((N, D), jnp.float32),
  )(agg2, deg2, x, W_l, b_l2, W_r)


@jax.jit
def kernel(x, edge_index, W_l, b_l, W_r):
  pad_src = jnp.broadcast_to(jnp.arange(EPW_PAD - EPW, dtype=jnp.int32),
                             (NW, EPW_PAD - EPW))
  src = jnp.concatenate([edge_index[0].reshape(NW, EPW), pad_src], axis=1)
  pad_dst = jnp.broadcast_to(N + jnp.arange(EPW_PAD - EPW, dtype=jnp.int32),
                             (NW, EPW_PAD - EPW))
  dst = jnp.concatenate([edge_index[1].reshape(NW, EPW), pad_dst], axis=1)
  src3 = src.reshape(NW, NCHUNK, CB)
  dst3 = dst.reshape(NW, NCHUNK, CB)
  zeros1 = jnp.zeros((NP, 8), jnp.float32)
  ones = jnp.zeros((CB, 8), jnp.float32).at[:, 0].set(1.0)
  agg2, deg2 = _sc_aggregate(x, src3, dst3, zeros1, ones)
  return _tc_finish(agg2, deg2, x, W_l, b_l.reshape(1, D), W_r)
